# baseline (speedup 1.0000x reference)
import jax
import jax.numpy as jnp
from jax.experimental import pallas as pl


_ROWS_PER_STEP = 1024
_GROUP = 128


def _dot_sigmoid_body(t_ref, c_ref, o_ref):
    p = t_ref[...] * c_ref[...]
    ngroups = _ROWS_PER_STEP // _GROUP
    p3 = p.reshape(ngroups, _GROUP, _GROUP)
    rows = []
    for g in range(ngroups):
        pt = p3[g].T
        rows.append(jnp.sum(pt, axis=0))
    o_ref[...] = jax.nn.sigmoid(jnp.stack(rows))


def kernel(target_embeds, context_embeds):
    batch, dim = target_embeds.shape
    nsteps = batch // _ROWS_PER_STEP
    ngroups = _ROWS_PER_STEP // _GROUP
    out2d = pl.pallas_call(
        _dot_sigmoid_body,
        grid=(nsteps,),
        in_specs=[
            pl.BlockSpec((_ROWS_PER_STEP, dim), lambda i: (i, 0)),
            pl.BlockSpec((_ROWS_PER_STEP, dim), lambda i: (i, 0)),
        ],
        out_specs=pl.BlockSpec((ngroups, _GROUP), lambda i: (i, 0)),
        out_shape=jax.ShapeDtypeStruct((batch // _GROUP, _GROUP), jnp.float32),
    )(target_embeds, context_embeds)
    return out2d.reshape(batch)


# speedup vs baseline: 1.7270x; 1.7270x over previous
import jax
import jax.numpy as jnp
from jax.experimental import pallas as pl


_ROWS_PER_STEP = 4096
_GROUP = 128


def _dot_sigmoid_body(t_ref, c_ref, o_ref):
    p = t_ref[...] * c_ref[...]
    ngroups = _ROWS_PER_STEP // _GROUP
    p3 = p.reshape(ngroups, _GROUP, _GROUP)
    rows = []
    for g in range(ngroups):
        pt = p3[g].T
        rows.append(jnp.sum(pt, axis=0))
    o_ref[...] = jax.nn.sigmoid(jnp.stack(rows))


def kernel(target_embeds, context_embeds):
    batch, dim = target_embeds.shape
    nsteps = batch // _ROWS_PER_STEP
    ngroups = _ROWS_PER_STEP // _GROUP
    out2d = pl.pallas_call(
        _dot_sigmoid_body,
        grid=(nsteps,),
        in_specs=[
            pl.BlockSpec((_ROWS_PER_STEP, dim), lambda i: (i, 0)),
            pl.BlockSpec((_ROWS_PER_STEP, dim), lambda i: (i, 0)),
        ],
        out_specs=pl.BlockSpec((ngroups, _GROUP), lambda i: (i, 0)),
        out_shape=jax.ShapeDtypeStruct((batch // _GROUP, _GROUP), jnp.float32),
    )(target_embeds, context_embeds)
    return out2d.reshape(batch)
